# Initial kernel scaffold; baseline (speedup 1.0000x reference)
import jax
import jax.numpy as jnp
from jax.experimental import pallas as pl


def kernel(x, edges, distance_matrix, w1, w2):
    raise NotImplementedError("write your pallas kernel here")


# speedup vs baseline: 49.6069x; 49.6069x over previous
import jax
import jax.numpy as jnp
from jax.experimental import pallas as pl

_N = 2048
_F = 16
_BLK = 256


def _gcn_block(x_ref, e_ref, d0_ref, d1_ref, w1x_ref, w1d_ref, w2_ref, o_ref):
    i = pl.program_id(0)
    x = x_ref[...]
    e = e_ref[...]
    xw = jnp.dot(x, w1x_ref[...], preferred_element_type=jnp.float32)
    agg = jnp.dot(e, xw, preferred_element_type=jnp.float32)
    b0 = jnp.sum(e * d0_ref[...], axis=1, keepdims=True)
    b1 = jnp.sum(e * d1_ref[...], axis=1, keepdims=True)
    w1d = w1d_ref[...]
    bc = b0 * w1d[0, :][None, :] + b1 * w1d[1, :][None, :]
    xi = x_ref[pl.ds(i * _BLK, _BLK), :]
    o_ref[...] = (
        jnp.dot(xi, w2_ref[...], preferred_element_type=jnp.float32) + agg + bc
    )


def kernel(x, edges, distance_matrix, w1, w2):
    w1x = w1[:, :_F].T
    w1d = jnp.zeros((8, _F), jnp.float32).at[:2].set(w1[:, _F:].T)
    w2t = w2.T
    d0 = distance_matrix[:, :, 0]
    d1 = distance_matrix[:, :, 1]

    grid = (_N // _BLK,)
    return pl.pallas_call(
        _gcn_block,
        grid=grid,
        in_specs=[
            pl.BlockSpec((_N, _F), lambda i: (0, 0)),
            pl.BlockSpec((_BLK, _N), lambda i: (i, 0)),
            pl.BlockSpec((_BLK, _N), lambda i: (i, 0)),
            pl.BlockSpec((_BLK, _N), lambda i: (i, 0)),
            pl.BlockSpec((_F, _F), lambda i: (0, 0)),
            pl.BlockSpec((8, _F), lambda i: (0, 0)),
            pl.BlockSpec((_F, _F), lambda i: (0, 0)),
        ],
        out_specs=pl.BlockSpec((_BLK, _F), lambda i: (i, 0)),
        out_shape=jax.ShapeDtypeStruct((_N, _F), jnp.float32),
    )(x, edges, d0, d1, w1x, w1d, w2t)
